# baseline (speedup 1.0000x reference)
import jax
import jax.numpy as jnp
from jax.experimental import pallas as pl


def _mlp_body(feats_ref, pts_ref, mask_ref, w1f_ref, w1p_ref, b1_ref,
              w2_ref, b2_ref, w3_ref, b3_ref, out_ref):
    feats = feats_ref[0]
    pts = pts_ref[0]
    mask = mask_ref[0]
    hp = jax.lax.Precision.HIGHEST
    h = jnp.dot(feats, w1f_ref[...], precision=hp,
                preferred_element_type=jnp.float32)
    h = h + jnp.dot(pts, w1p_ref[...], precision=hp,
                    preferred_element_type=jnp.float32)
    h = jnp.maximum(h + b1_ref[...], 0.0)
    h = jnp.maximum(jnp.dot(h, w2_ref[...], precision=hp,
                            preferred_element_type=jnp.float32) + b2_ref[...], 0.0)
    s = jax.nn.sigmoid(jnp.dot(h, w3_ref[...], precision=hp,
                               preferred_element_type=jnp.float32) + b3_ref[...])
    scores = s * mask
    tot = jnp.sum(mask)
    out_ref[0] = jnp.where(tot < 10.0, jnp.zeros_like(scores), scores)


def kernel(points, features, leaf_mask, W1, b1, W2, b2, W3, b3):
    B, N, F = features.shape
    W1f = W1[:F]
    W1p = W1[F:]
    b1r = b1.reshape(1, -1)
    b2r = b2.reshape(1, -1)
    b3r = b3.reshape(1, -1)
    mask3 = leaf_mask.reshape(B, N, 1)

    out = pl.pallas_call(
        _mlp_body,
        grid=(B,),
        in_specs=[
            pl.BlockSpec((1, N, F), lambda b: (b, 0, 0)),
            pl.BlockSpec((1, N, 3), lambda b: (b, 0, 0)),
            pl.BlockSpec((1, N, 1), lambda b: (b, 0, 0)),
            pl.BlockSpec(W1f.shape, lambda b: (0, 0)),
            pl.BlockSpec(W1p.shape, lambda b: (0, 0)),
            pl.BlockSpec(b1r.shape, lambda b: (0, 0)),
            pl.BlockSpec(W2.shape, lambda b: (0, 0)),
            pl.BlockSpec(b2r.shape, lambda b: (0, 0)),
            pl.BlockSpec(W3.shape, lambda b: (0, 0)),
            pl.BlockSpec(b3r.shape, lambda b: (0, 0)),
        ],
        out_specs=pl.BlockSpec((1, N, 1), lambda b: (b, 0, 0)),
        out_shape=jax.ShapeDtypeStruct((B, N, 1), jnp.float32),
    )(features, points, mask3, W1f, W1p, b1r, W2, b2r, W3, b3r)
    return out.reshape(B, N)


# speedup vs baseline: 3.5217x; 3.5217x over previous
import jax
import jax.numpy as jnp
from jax.experimental import pallas as pl


def _mlp_body(ft_ref, pt_ref, mask_ref, w1ft_ref, w1pt_ref, b1_ref,
              w2t_ref, b2_ref, w3t_ref, b3_ref, out_ref):
    ft = ft_ref[0]
    pt = pt_ref[0]
    h = jnp.dot(w1ft_ref[...], ft, preferred_element_type=jnp.float32)
    h = h + jnp.dot(w1pt_ref[...], pt, preferred_element_type=jnp.float32)
    h = jnp.maximum(h + b1_ref[...], 0.0)
    h = jnp.maximum(jnp.dot(w2t_ref[...], h,
                            preferred_element_type=jnp.float32) + b2_ref[...], 0.0)
    z = jnp.dot(w3t_ref[...], h, preferred_element_type=jnp.float32) + b3_ref[...]
    s = jax.nn.sigmoid(z)
    m = mask_ref[0]
    sc = s * m
    out_ref[0] = jnp.where(jnp.sum(m) < 10.0, jnp.zeros_like(sc), sc)


def kernel(points, features, leaf_mask, W1, b1, W2, b2, W3, b3):
    B, N, F = features.shape
    ft = features.transpose(0, 2, 1)
    ptp = jnp.concatenate(
        [points, jnp.zeros((B, N, 5), points.dtype)], -1).transpose(0, 2, 1)
    W1ft = W1[:F].T
    W1pt = jnp.concatenate(
        [W1[F:], jnp.zeros((5, W1.shape[1]), W1.dtype)], 0).T
    W2t = W2.T
    W3t = W3.T
    b1c = b1.reshape(-1, 1)
    b2c = b2.reshape(-1, 1)
    b3c = b3.reshape(-1, 1)
    mask_r = leaf_mask.reshape(B, 1, N)

    out = pl.pallas_call(
        _mlp_body,
        grid=(B,),
        in_specs=[
            pl.BlockSpec((1, F, N), lambda b: (b, 0, 0)),
            pl.BlockSpec((1, 8, N), lambda b: (b, 0, 0)),
            pl.BlockSpec((1, 1, N), lambda b: (b, 0, 0)),
            pl.BlockSpec(W1ft.shape, lambda b: (0, 0)),
            pl.BlockSpec(W1pt.shape, lambda b: (0, 0)),
            pl.BlockSpec(b1c.shape, lambda b: (0, 0)),
            pl.BlockSpec(W2t.shape, lambda b: (0, 0)),
            pl.BlockSpec(b2c.shape, lambda b: (0, 0)),
            pl.BlockSpec(W3t.shape, lambda b: (0, 0)),
            pl.BlockSpec(b3c.shape, lambda b: (0, 0)),
        ],
        out_specs=pl.BlockSpec((1, 1, N), lambda b: (b, 0, 0)),
        out_shape=jax.ShapeDtypeStruct((B, 1, N), jnp.float32),
    )(ft, ptp, mask_r, W1ft, W1pt, b1c, W2t, b2c, W3t, b3c)
    return out.reshape(B, N)
